# baseline (speedup 1.0000x reference)
import functools
import math

import jax
import jax.numpy as jnp
from jax import lax
from jax.experimental import pallas as pl
from jax.experimental.pallas import tpu as pltpu
from jax.experimental.pallas import tpu_sc as plsc

VOCAB = 1000000
NUM_WAVES = 16
D_MODEL = 64
B, T = 4096, 50
N = B * T
CH = 128

_info = plsc.get_sparse_core_info()
NC, NS = _info.num_cores, _info.num_subcores
NW = NC * NS
PER_W = N // NW
CHUNKS = PER_W // CH

TWO_PI = float(2.0 * math.pi)


CROWS = CH * NUM_WAVES // 128

CVT_COLS = 512
CVT_OROWS = CVT_COLS * NUM_WAVES // 128
FULL_TILES = (VOCAB // 128) // 1 - 0
TILES_PER_W = 244
CHUNK_TILES = CVT_COLS // 128
CHUNKS_PER_W = TILES_PER_W // CHUNK_TILES
TAIL_COLS = VOCAB - 7808 * 128
OUT_ROWS_TBL = VOCAB * NUM_WAVES // 128


def _sc_convert():
    mesh = plsc.VectorSubcoreMesh(core_axis_name="c", subcore_axis_name="s")
    out = jax.ShapeDtypeStruct((OUT_ROWS_TBL, 128), jnp.float32)
    ibuf = pltpu.VMEM((NUM_WAVES, CVT_COLS), jnp.float32)
    obuf = pltpu.VMEM((CVT_OROWS, 128), jnp.float32)

    @functools.partial(
        pl.kernel,
        mesh=mesh,
        out_type=[out, out, out],
        compiler_params=pltpu.CompilerParams(needs_layout_passes=False),
        scratch_types=[
            pltpu.VMEM((2, NUM_WAVES, CVT_COLS), jnp.float32),
            pltpu.VMEM((2, CVT_OROWS, 128), jnp.float32),
            pltpu.SemaphoreType.DMA,
            pltpu.SemaphoreType.DMA,
            pltpu.SemaphoreType.DMA,
            pltpu.SemaphoreType.DMA,
        ],
    )
    def convert_k(fr_hbm, ph_hbm, am_hbm, fr_tail, ph_tail, am_tail,
                  fr_out, ph_out, am_out, ib, ob, gs0, gs1, ws0, ws1):
        wid = lax.axis_index("s") * NC + lax.axis_index("c")
        wlane = lax.broadcasted_iota(jnp.int32, (16,), 0)
        gsems = (gs0, gs1)
        wsems = (ws0, ws1)

        def c0_of(i):
            return (wid * TILES_PER_W + i * CHUNK_TILES) * 128

        def in_copy(src, i, s):
            return pltpu.make_async_copy(
                src.at[:, pl.ds(pl.multiple_of(c0_of(i), 128), CVT_COLS)],
                ib.at[s], gsems[s])

        def out_copy(dst, i, s):
            return pltpu.make_async_copy(
                ob.at[s],
                dst.at[pl.ds(pl.multiple_of(c0_of(i) // 8, 8), CVT_OROWS)],
                wsems[s])

        def repack(s):
            @plsc.parallel_loop(0, CVT_OROWS, 1, unroll=4)
            def row_body(rl):
                base = rl * 8
                vals = [plsc.load_gather(
                            ib.at[s], [wlane,
                                       jnp.full((16,), base + k, jnp.int32)])
                        for k in range(8)]
                for k in range(8):
                    ob[s, rl, pl.ds(k * 16, 16)] = vals[k]

        for src, dst in ((fr_hbm, fr_out), (ph_hbm, ph_out), (am_hbm, am_out)):
            in_copy(src, 0, 0).start()

            def body(jj, cr, src=src, dst=dst):
                j0 = 2 * jj
                j1 = j0 + 1
                in_copy(src, j1, 1).start()
                in_copy(src, j0, 0).wait()

                @pl.when(jj >= 1)
                def _():
                    out_copy(dst, j0 - 2, 0).wait()

                repack(0)
                out_copy(dst, j0, 0).start()

                in_copy(src, j1 + 1, 0).start()
                in_copy(src, j1, 1).wait()

                @pl.when(jj >= 1)
                def _():
                    out_copy(dst, j1 - 2, 1).wait()

                repack(1)
                out_copy(dst, j1, 1).start()
                return cr

            lax.fori_loop(0, CHUNKS_PER_W // 2, body, 0)
            last = CHUNKS_PER_W - 1
            in_copy(src, last, 0).wait()
            out_copy(dst, last - 2, 0).wait()
            repack(0)
            out_copy(dst, last, 0).start()
            out_copy(dst, last - 1, 1).wait()
            out_copy(dst, last, 0).wait()

        @pl.when(wid == NW - 1)
        def _():
            for src, tail, dst in ((fr_hbm, fr_tail, fr_out),
                                   (ph_hbm, ph_tail, ph_out),
                                   (am_hbm, am_tail, am_out)):
                pltpu.sync_copy(src.at[:, pl.ds(7808 * 128, CVT_COLS)],
                                ib.at[0])
                repack(0)
                pltpu.sync_copy(ob.at[0], dst.at[pl.ds(124928, CVT_OROWS)])
                pltpu.sync_copy(tail, ob.at[0, pl.ds(0, 8)])
                pltpu.sync_copy(ob.at[0, pl.ds(0, 8)],
                                dst.at[pl.ds(124992, 8)])

    return convert_k


_convert = _sc_convert()


def _sc_gather():
    mesh = plsc.VectorSubcoreMesh(core_axis_name="c", subcore_axis_name="s")
    out = jax.ShapeDtypeStruct((N // 8, 128), jnp.float32)
    gbuf = pltpu.VMEM((CH, NUM_WAVES), jnp.float32)

    cbuf = pltpu.VMEM((CROWS, 128), jnp.float32)

    @functools.partial(
        pl.kernel,
        mesh=mesh,
        out_type=[out, out, out],
        compiler_params=pltpu.CompilerParams(use_tc_tiling_on_sc=False),
        scratch_types=[
            pltpu.VMEM((PER_W,), jnp.int32),
            gbuf, gbuf, gbuf,
            cbuf, cbuf, cbuf,
            pltpu.SemaphoreType.DMA,
        ],
    )
    def gather_k(ids_hbm, fr_hbm, ph_hbm, am_hbm,
                 fr_out, ph_out, am_out,
                 idx_v, g0, g1, g2, c0, c1, c2, sem):
        wid = lax.axis_index("s") * NC + lax.axis_index("c")
        pltpu.sync_copy(ids_hbm.at[pl.ds(wid * PER_W, PER_W)], idx_v)

        def body(j, carry):
            idx = idx_v.at[pl.ds(j * CH, CH)]
            d0 = pltpu.async_copy(fr_hbm.at[idx], g0, sem)
            d1 = pltpu.async_copy(ph_hbm.at[idx], g1, sem)
            d2 = pltpu.async_copy(am_hbm.at[idx], g2, sem)
            d0.wait()
            d1.wait()
            d2.wait()

            def row_body(i, _):
                for k in range(8):
                    t = i * 8 + k
                    c0[i, pl.ds(k * 16, 16)] = g0[t, :]
                    c1[i, pl.ds(k * 16, 16)] = g1[t, :]
                    c2[i, pl.ds(k * 16, 16)] = g2[t, :]
                return 0

            lax.fori_loop(0, CROWS, row_body, 0)

            row = wid * (PER_W // 8) + j * CROWS
            pltpu.sync_copy(c0, fr_out.at[pl.ds(row, CROWS)])
            pltpu.sync_copy(c1, ph_out.at[pl.ds(row, CROWS)])
            pltpu.sync_copy(c2, am_out.at[pl.ds(row, CROWS)])
            return carry

        lax.fori_loop(0, CHUNKS, body, 0)

    return gather_k


_gather = _sc_gather()

BB = 800
ROWS = N // 8
GRID = ROWS // BB


BS = BB * 8 // T


def _tc_body(fr_ref, ph_ref, am_ref, pp_ref, wb_ref, bb_ref, out_ref):
    pp = jnp.tile(pp_ref[...], (BB // 25, 1))
    wp = fr_ref[...] * TWO_PI + ph_ref[...] + pp
    am = am_ref[...]
    sw = am * jnp.sin(wp)
    cw = am * jnp.cos(wp)
    x = jnp.concatenate([sw, cw], axis=1)
    y = jnp.dot(x, wb_ref[...], preferred_element_type=jnp.float32,
                precision=lax.Precision.HIGHEST)
    out_ref[...] = y + bb_ref[...]


def _tc_compute(fr, ph, am, pp_table, w_big, b_big):
    blk = lambda i: (i, 0)
    const = lambda i: (0, 0)
    return pl.pallas_call(
        _tc_body,
        grid=(GRID,),
        in_specs=[
            pl.BlockSpec((BB, 128), blk),
            pl.BlockSpec((BB, 128), blk),
            pl.BlockSpec((BB, 128), blk),
            pl.BlockSpec((25, 128), const),
            pl.BlockSpec((256, 512), const),
            pl.BlockSpec((1, 512), const),
        ],
        out_specs=pl.BlockSpec((BB, 512), blk),
        out_shape=jax.ShapeDtypeStruct((ROWS, 512), jnp.float32),
    )(fr, ph, am, pp_table, w_big, b_big)


def kernel(token_ids, token_freqs, token_phases, token_amps, W, b, pos_freq):
    ids1d = token_ids.reshape(N).astype(jnp.int32)
    tails = [t[999936:].reshape(8, 128) for t in
             (token_freqs, token_phases, token_amps)]
    frl, phl, aml = _convert(token_freqs.T, token_phases.T, token_amps.T,
                             *tails)
    fr, ph, am = _gather(ids1d,
                         frl.reshape(VOCAB, NUM_WAVES),
                         phl.reshape(VOCAB, NUM_WAVES),
                         aml.reshape(VOCAB, NUM_WAVES))

    pos = (jnp.arange(200, dtype=jnp.float32) % T).reshape(200, 1)
    pp_table = (pos * pos_freq.reshape(1, NUM_WAVES)).reshape(25, 128)

    eye8 = jnp.eye(8, dtype=jnp.float32)
    w_big = jnp.concatenate(
        [jnp.kron(eye8, W[:NUM_WAVES]), jnp.kron(eye8, W[NUM_WAVES:])], axis=0)
    b_big = jnp.tile(b, 8).reshape(1, 512)

    out = _tc_compute(fr, ph, am, pp_table, w_big, b_big)
    return out.reshape(B, T, D_MODEL)


# speedup vs baseline: 1.0452x; 1.0452x over previous
import functools
import math

import jax
import jax.numpy as jnp
from jax import lax
from jax.experimental import pallas as pl
from jax.experimental.pallas import tpu as pltpu
from jax.experimental.pallas import tpu_sc as plsc

VOCAB = 1000000
NUM_WAVES = 16
D_MODEL = 64
B, T = 4096, 50
N = B * T
CH = 128

_info = plsc.get_sparse_core_info()
NC, NS = _info.num_cores, _info.num_subcores
NW = NC * NS
PER_W = N // NW
CHUNKS = PER_W // CH

TWO_PI = float(2.0 * math.pi)


CROWS = CH * NUM_WAVES // 128

CVT_COLS = 512
CVT_OROWS = CVT_COLS * NUM_WAVES // 128
FULL_TILES = (VOCAB // 128) // 1 - 0
TILES_PER_W = 244
CHUNK_TILES = CVT_COLS // 128
CHUNKS_PER_W = TILES_PER_W // CHUNK_TILES
TAIL_COLS = VOCAB - 7808 * 128
OUT_ROWS_TBL = VOCAB * NUM_WAVES // 128


def _sc_convert():
    mesh = plsc.VectorSubcoreMesh(core_axis_name="c", subcore_axis_name="s")
    out = jax.ShapeDtypeStruct((OUT_ROWS_TBL, 128), jnp.float32)
    ibuf = pltpu.VMEM((NUM_WAVES, CVT_COLS), jnp.float32)
    obuf = pltpu.VMEM((CVT_OROWS, 128), jnp.float32)

    @functools.partial(
        pl.kernel,
        mesh=mesh,
        out_type=[out, out, out],
        compiler_params=pltpu.CompilerParams(needs_layout_passes=False),
        scratch_types=[
            pltpu.VMEM((2, NUM_WAVES, CVT_COLS), jnp.float32),
            pltpu.VMEM((2, CVT_OROWS, 128), jnp.float32),
            pltpu.SemaphoreType.DMA,
            pltpu.SemaphoreType.DMA,
            pltpu.SemaphoreType.DMA,
            pltpu.SemaphoreType.DMA,
        ],
    )
    def convert_k(fr_hbm, ph_hbm, am_hbm, fr_tail, ph_tail, am_tail,
                  fr_out, ph_out, am_out, ib, ob, gs0, gs1, ws0, ws1):
        wid = lax.axis_index("s") * NC + lax.axis_index("c")
        wlane = lax.broadcasted_iota(jnp.int32, (16,), 0)
        gsems = (gs0, gs1)
        wsems = (ws0, ws1)

        def c0_of(i):
            return (wid * TILES_PER_W + i * CHUNK_TILES) * 128

        def in_copy(src, i, s):
            return pltpu.make_async_copy(
                src.at[:, pl.ds(pl.multiple_of(c0_of(i), 128), CVT_COLS)],
                ib.at[s], gsems[s])

        def out_copy(dst, i, s):
            return pltpu.make_async_copy(
                ob.at[s],
                dst.at[pl.ds(pl.multiple_of(c0_of(i) // 8, 8), CVT_OROWS)],
                wsems[s])

        def repack(s):
            @plsc.parallel_loop(0, CVT_OROWS, 1, unroll=4)
            def row_body(rl):
                base = rl * 8
                vals = [plsc.load_gather(
                            ib.at[s], [wlane,
                                       jnp.full((16,), base + k, jnp.int32)])
                        for k in range(8)]
                for k in range(8):
                    ob[s, rl, pl.ds(k * 16, 16)] = vals[k]

        for src, dst in ((fr_hbm, fr_out), (ph_hbm, ph_out), (am_hbm, am_out)):
            in_copy(src, 0, 0).start()

            def body(jj, cr, src=src, dst=dst):
                j0 = 2 * jj
                j1 = j0 + 1
                in_copy(src, j1, 1).start()
                in_copy(src, j0, 0).wait()

                @pl.when(jj >= 1)
                def _():
                    out_copy(dst, j0 - 2, 0).wait()

                repack(0)
                out_copy(dst, j0, 0).start()

                in_copy(src, j1 + 1, 0).start()
                in_copy(src, j1, 1).wait()

                @pl.when(jj >= 1)
                def _():
                    out_copy(dst, j1 - 2, 1).wait()

                repack(1)
                out_copy(dst, j1, 1).start()
                return cr

            lax.fori_loop(0, CHUNKS_PER_W // 2, body, 0)
            last = CHUNKS_PER_W - 1
            in_copy(src, last, 0).wait()
            out_copy(dst, last - 2, 0).wait()
            repack(0)
            out_copy(dst, last, 0).start()
            out_copy(dst, last - 1, 1).wait()
            out_copy(dst, last, 0).wait()

        @pl.when(wid == NW - 1)
        def _():
            for src, tail, dst in ((fr_hbm, fr_tail, fr_out),
                                   (ph_hbm, ph_tail, ph_out),
                                   (am_hbm, am_tail, am_out)):
                pltpu.sync_copy(src.at[:, pl.ds(7808 * 128, CVT_COLS)],
                                ib.at[0])
                repack(0)
                pltpu.sync_copy(ob.at[0], dst.at[pl.ds(124928, CVT_OROWS)])
                pltpu.sync_copy(tail, ob.at[0, pl.ds(0, 8)])
                pltpu.sync_copy(ob.at[0, pl.ds(0, 8)],
                                dst.at[pl.ds(124992, 8)])

    return convert_k


_convert = _sc_convert()


def _sc_gather():
    mesh = plsc.VectorSubcoreMesh(core_axis_name="c", subcore_axis_name="s")
    out = jax.ShapeDtypeStruct((N // 8, 128), jnp.float32)
    gbuf = pltpu.VMEM((CH, NUM_WAVES), jnp.float32)

    cbuf = pltpu.VMEM((CROWS, 128), jnp.float32)

    @functools.partial(
        pl.kernel,
        mesh=mesh,
        out_type=[out, out, out],
        compiler_params=pltpu.CompilerParams(use_tc_tiling_on_sc=False),
        scratch_types=[
            pltpu.VMEM((PER_W,), jnp.int32),
            gbuf, gbuf, gbuf,
            cbuf, cbuf, cbuf,
            pltpu.SemaphoreType.DMA,
        ],
    )
    def gather_k(ids_hbm, fr_hbm, ph_hbm, am_hbm,
                 fr_out, ph_out, am_out,
                 idx_v, g0, g1, g2, c0, c1, c2, sem):
        wid = lax.axis_index("s") * NC + lax.axis_index("c")
        pltpu.sync_copy(ids_hbm.at[pl.ds(wid * PER_W, PER_W)], idx_v)

        def body(j, carry):
            idx = idx_v.at[pl.ds(j * CH, CH)]
            d0 = pltpu.async_copy(fr_hbm.at[idx], g0, sem)
            d1 = pltpu.async_copy(ph_hbm.at[idx], g1, sem)
            d2 = pltpu.async_copy(am_hbm.at[idx], g2, sem)
            d0.wait()
            d1.wait()
            d2.wait()

            def row_body(i, _):
                for k in range(8):
                    t = i * 8 + k
                    c0[i, pl.ds(k * 16, 16)] = g0[t, :]
                    c1[i, pl.ds(k * 16, 16)] = g1[t, :]
                    c2[i, pl.ds(k * 16, 16)] = g2[t, :]
                return 0

            lax.fori_loop(0, CROWS, row_body, 0)

            row = wid * (PER_W // 8) + j * CROWS
            pltpu.sync_copy(c0, fr_out.at[pl.ds(row, CROWS)])
            pltpu.sync_copy(c1, ph_out.at[pl.ds(row, CROWS)])
            pltpu.sync_copy(c2, am_out.at[pl.ds(row, CROWS)])
            return carry

        lax.fori_loop(0, CHUNKS, body, 0)

    return gather_k


_gather = _sc_gather()

BB = 1600
ROWS = N // 8
GRID = ROWS // BB


BS = BB * 8 // T


def _tc_body(fr_ref, ph_ref, am_ref, pp_ref, wb_ref, bb_ref, out_ref):
    pp = jnp.tile(pp_ref[...], (BB // 25, 1))
    wp = fr_ref[...] * TWO_PI + ph_ref[...] + pp
    am = am_ref[...]
    sw = am * jnp.sin(wp)
    cw = am * jnp.cos(wp)
    x = jnp.concatenate([sw, cw], axis=1)
    y = jnp.dot(x, wb_ref[...], preferred_element_type=jnp.float32)
    out_ref[...] = y + bb_ref[...]


def _tc_compute(fr, ph, am, pp_table, w_big, b_big):
    blk = lambda i: (i, 0)
    const = lambda i: (0, 0)
    return pl.pallas_call(
        _tc_body,
        grid=(GRID,),
        in_specs=[
            pl.BlockSpec((BB, 128), blk),
            pl.BlockSpec((BB, 128), blk),
            pl.BlockSpec((BB, 128), blk),
            pl.BlockSpec((25, 128), const),
            pl.BlockSpec((256, 512), const),
            pl.BlockSpec((1, 512), const),
        ],
        out_specs=pl.BlockSpec((BB, 512), blk),
        out_shape=jax.ShapeDtypeStruct((ROWS, 512), jnp.float32),
    )(fr, ph, am, pp_table, w_big, b_big)


def kernel(token_ids, token_freqs, token_phases, token_amps, W, b, pos_freq):
    ids1d = token_ids.reshape(N).astype(jnp.int32)
    tails = [t[999936:].reshape(8, 128) for t in
             (token_freqs, token_phases, token_amps)]
    frl, phl, aml = _convert(token_freqs.T, token_phases.T, token_amps.T,
                             *tails)
    fr, ph, am = _gather(ids1d,
                         frl.reshape(VOCAB, NUM_WAVES),
                         phl.reshape(VOCAB, NUM_WAVES),
                         aml.reshape(VOCAB, NUM_WAVES))

    pos = (jnp.arange(200, dtype=jnp.float32) % T).reshape(200, 1)
    pp_table = (pos * pos_freq.reshape(1, NUM_WAVES)).reshape(25, 128)

    eye8 = jnp.eye(8, dtype=jnp.float32)
    w_big = jnp.concatenate(
        [jnp.kron(eye8, W[:NUM_WAVES]), jnp.kron(eye8, W[NUM_WAVES:])], axis=0)
    b_big = jnp.tile(b, 8).reshape(1, 512)

    out = _tc_compute(fr, ph, am, pp_table, w_big, b_big)
    return out.reshape(B, T, D_MODEL)
